# baseline (speedup 1.0000x reference)
import math

import jax
import jax.numpy as jnp
from jax.experimental import pallas as pl
from jax.experimental.pallas import tpu as pltpu

VOCAB = 32000
SMOOTH = 0.1
CONF = 1.0 - SMOOTH
SV = SMOOTH / (VOCAB - 2 + 1e-06)
LOG_SV = math.log(SV)
LOG_CONF = math.log(CONF)
ENT_BASE = (VOCAB - 2) * SV * LOG_SV
ENT_DELTA = CONF * LOG_CONF - SV * LOG_SV

RB = 128
VB = 16000


def _loss_body(x_ref, tgt_ref, out_ref):
    i = pl.program_id(0)
    j = pl.program_id(1)
    nr = pl.num_programs(0)
    nv = pl.num_programs(1)

    @pl.when((i == 0) & (j == 0))
    def _init():
        out_ref[...] = jnp.zeros_like(out_ref)

    blk = x_ref[...]
    tgt = tgt_ref[0, pl.ds(i * RB, RB)]
    tmask = jnp.where(tgt >= 2, tgt, -1)
    tloc = (tmask - j * VB)[:, None]
    cols = jax.lax.broadcasted_iota(jnp.int32, (RB, VB), 1)
    hit_sum = jnp.sum(jnp.where(cols == tloc, blk, 0.0))
    acc = -SV * jnp.sum(blk) - (CONF - SV) * hit_sum

    @pl.when(j == 0)
    def _edge():
        out_ref[...] = out_ref[...] + SV * jnp.sum(blk[:, 0] + blk[:, 1])

    @pl.when((i == nr - 1) & (j == nv - 1))
    def _entropy():
        t_all = tgt_ref[0, :]
        cnt = jnp.sum(jnp.where(t_all >= 2, 1.0, 0.0))
        out_ref[...] = out_ref[...] + (t_all.shape[0] * ENT_BASE + cnt * ENT_DELTA)

    out_ref[...] = out_ref[...] + acc


def kernel(x, target):
    rows = x.shape[0] * x.shape[1]
    x2d = x.reshape(rows, VOCAB)
    tgt2d = target.reshape(1, rows)
    nr = rows // RB
    nv = VOCAB // VB
    out = pl.pallas_call(
        _loss_body,
        grid=(nr, nv),
        in_specs=[
            pl.BlockSpec((RB, VB), lambda i, j: (i, j)),
            pl.BlockSpec((1, rows), lambda i, j: (0, 0)),
        ],
        out_specs=pl.BlockSpec((1, 1), lambda i, j: (0, 0)),
        out_shape=jax.ShapeDtypeStruct((1, 1), jnp.float32),
        compiler_params=pltpu.CompilerParams(vmem_limit_bytes=128 * 1024 * 1024),
    )(x2d, tgt2d)
    return out[0, 0]


# speedup vs baseline: 1.2182x; 1.2182x over previous
import math

import jax
import jax.numpy as jnp
from jax.experimental import pallas as pl

VOCAB = 32000
SMOOTH = 0.1
CONF = 1.0 - SMOOTH
SV = SMOOTH / (VOCAB - 2 + 1e-06)
LOG_SV = math.log(SV)
LOG_CONF = math.log(CONF)
ENT_BASE = (VOCAB - 2) * SV * LOG_SV
ENT_DELTA = CONF * LOG_CONF - SV * LOG_SV

RB = 128
VB = 32000


def _loss_body(x_ref, tgt_ref, out_ref):
    i = pl.program_id(0)
    nr = pl.num_programs(0)

    @pl.when(i == 0)
    def _init():
        out_ref[...] = jnp.zeros_like(out_ref)

    blk = x_ref[...]
    tgt = tgt_ref[0, pl.ds(i * RB, RB)]
    tloc = tgt[:, None]
    scale = jnp.where(tgt[:, None] >= 2, CONF / SV, 1.0)
    cols = jax.lax.broadcasted_iota(jnp.int32, (RB, VB), 1)
    val = jnp.where(cols == tloc, blk * scale, blk)
    acc = -SV * jnp.sum(val) + SV * jnp.sum(blk[:, 0] + blk[:, 1])

    @pl.when(i == nr - 1)
    def _entropy():
        t_all = tgt_ref[0, :]
        cnt = jnp.sum(jnp.where(t_all >= 2, 1.0, 0.0))
        out_ref[...] = out_ref[...] + (t_all.shape[0] * ENT_BASE + cnt * ENT_DELTA)

    out_ref[...] = out_ref[...] + acc


def kernel(x, target):
    rows = x.shape[0] * x.shape[1]
    x2d = x.reshape(rows, VOCAB)
    tgt2d = target.reshape(1, rows)
    nr = rows // RB
    out = pl.pallas_call(
        _loss_body,
        grid=(nr,),
        in_specs=[
            pl.BlockSpec((RB, VB), lambda i: (i, 0)),
            pl.BlockSpec((1, rows), lambda i: (0, 0)),
        ],
        out_specs=pl.BlockSpec((1, 1), lambda i: (0, 0)),
        out_shape=jax.ShapeDtypeStruct((1, 1), jnp.float32),
    )(x2d, tgt2d)
    return out[0, 0]


# speedup vs baseline: 1.2620x; 1.0360x over previous
import math

import jax
import jax.numpy as jnp
from jax.experimental import pallas as pl
from jax.experimental.pallas import tpu as pltpu

VOCAB = 32000
SMOOTH = 0.1
CONF = 1.0 - SMOOTH
SV = SMOOTH / (VOCAB - 2 + 1e-06)
LOG_SV = math.log(SV)
LOG_CONF = math.log(CONF)
ENT_BASE = (VOCAB - 2) * SV * LOG_SV
ENT_DELTA = CONF * LOG_CONF - SV * LOG_SV

RB = 256
VB = 32000


def _loss_body(x_ref, tgt_ref, out_ref):
    i = pl.program_id(0)
    nr = pl.num_programs(0)

    @pl.when(i == 0)
    def _init():
        out_ref[...] = jnp.zeros_like(out_ref)

    blk = x_ref[...]
    tgt = tgt_ref[0, pl.ds(i * RB, RB)]
    tloc = tgt[:, None]
    scale = jnp.where(tgt[:, None] >= 2, CONF / SV, 1.0)
    cols = jax.lax.broadcasted_iota(jnp.int32, (RB, VB), 1)
    val = jnp.where(cols == tloc, blk * scale, blk)
    acc = -SV * jnp.sum(val) + SV * jnp.sum(blk[:, 0] + blk[:, 1])

    @pl.when(i == nr - 1)
    def _entropy():
        t_all = tgt_ref[0, :]
        cnt = jnp.sum(jnp.where(t_all >= 2, 1.0, 0.0))
        out_ref[...] = out_ref[...] + (t_all.shape[0] * ENT_BASE + cnt * ENT_DELTA)

    out_ref[...] = out_ref[...] + acc


def kernel(x, target):
    rows = x.shape[0] * x.shape[1]
    x2d = x.reshape(rows, VOCAB)
    tgt2d = target.reshape(1, rows)
    nr = rows // RB
    out = pl.pallas_call(
        _loss_body,
        grid=(nr,),
        in_specs=[
            pl.BlockSpec((RB, VB), lambda i: (i, 0)),
            pl.BlockSpec((1, rows), lambda i: (0, 0)),
        ],
        out_specs=pl.BlockSpec((1, 1), lambda i: (0, 0)),
        out_shape=jax.ShapeDtypeStruct((1, 1), jnp.float32),
        compiler_params=pltpu.CompilerParams(vmem_limit_bytes=128 * 1024 * 1024),
    )(x2d, tgt2d)
    return out[0, 0]
